# baseline (speedup 1.0000x reference)
import functools

import jax
import jax.numpy as jnp
from jax import lax
from jax.experimental import pallas as pl
from jax.experimental.pallas import tpu as pltpu
from jax.experimental.pallas import tpu_sc as plsc


def _leaky_relu(x, a=0.2):
    return jnp.where(x >= 0, x, a * x)


def _conv1d(x, w):
    return lax.conv_general_dilated(
        x, w, window_strides=(1,), padding=((1, 1),),
        dimension_numbers=('NCH', 'OIH', 'NCH'))


def _conv2d(x, w):
    return lax.conv_general_dilated(
        x, w, window_strides=(1, 1), padding=((1, 1), (1, 1)),
        dimension_numbers=('NCHW', 'OIHW', 'NCHW'))


def _batchnorm(x, g, b, eps=1e-5):
    m = jnp.mean(x, axis=(0, 2, 3), keepdims=True)
    v = jnp.var(x, axis=(0, 2, 3), keepdims=True)
    return (x - m) / jnp.sqrt(v + eps) * g.reshape(1, -1, 1, 1) + b.reshape(1, -1, 1, 1)


def _layernorm(x, g, b, eps=1e-5):
    m = jnp.mean(x, axis=-1, keepdims=True)
    v = jnp.var(x, axis=-1, keepdims=True)
    return (x - m) / jnp.sqrt(v + eps) * g + b


NUM_TYPES = 4


def _score_topk_kernel(rep_ref, filt_ref, sti_ref, rank_ref, *refs,
                       F, V, D, K, GB):
    T = NUM_TYPES
    G = GB * T
    bank_refs = refs[:G]
    fidx_ref = refs[G]
    i = pl.program_id(0)
    scores = []
    for g in range(GB):
        fv = filt_ref[g, 0]
        fm = jnp.broadcast_to(fv.astype(jnp.bfloat16)[:, None], (D, 128))
        for t in range(T):
            x = bank_refs[g * T + t][0, 0]
            n2 = jnp.sum(x * x, axis=1)
            n = jnp.maximum(jnp.sqrt(n2), 1e-12)
            xb = (x / n[:, None]).astype(jnp.bfloat16)
            d = jnp.dot(xb, fm, preferred_element_type=jnp.float32)
            scores.append(d[:, 0])
    R = V // 128
    s4 = jnp.stack(scores).reshape(G, R, 128)
    flat_i = jnp.broadcast_to(
        (lax.broadcasted_iota(jnp.int32, (R, 128), 0) * 128
         + lax.broadcasted_iota(jnp.int32, (R, 128), 1))[None],
        (G, R, 128))
    sti_rows = sti_ref[:, 0]
    rank_rows = rank_ref[:, 0]
    neg_inf = jnp.float32(-jnp.inf)
    big = jnp.int32(2 ** 30)
    maxc = jnp.int32(0)
    for t in range(T):
        cnt = jnp.sum((sti_rows == t).astype(jnp.int32), axis=1)
        maxc = jnp.maximum(maxc, jnp.max(cnt))

    def body(j, carry):
        s4, accs = carry
        m4 = jnp.max(s4, axis=(1, 2), keepdims=True)
        idx4 = jnp.min(jnp.where(s4 == m4, flat_i, big),
                       axis=(1, 2), keepdims=True)
        new_accs = []
        for g in range(GB):
            acc = accs[g]
            sel_j = rank_rows[g:g + 1] == j
            for t in range(T):
                acc = jnp.where(sel_j & (sti_rows[g:g + 1] == t),
                                idx4[g * T + t, 0, 0], acc)
            new_accs.append(acc)
        s4 = jnp.where(flat_i == idx4, neg_inf, s4)
        return s4, tuple(new_accs)

    _, accs = lax.fori_loop(
        0, maxc, body,
        (s4, tuple(jnp.zeros((1, F), jnp.int32) for _ in range(GB))))
    for g in range(GB):
        b = i * GB + g
        base = jnp.zeros((1, F), jnp.int32)
        for t in range(T):
            base = jnp.where(sti_rows[g:g + 1] == t,
                             (b * F + rep_ref[b, t]) * V, base)
        fidx_ref[g] = base + accs[g]


def _sc_gather_kernel(bank_ref, fidx_ref, out_ref, idx_v, rows_v, sem,
                      *, B, F, NC):
    c = lax.axis_index("c")
    s = lax.axis_index("s")
    wid = s * NC + c

    @pl.when(wid < B)
    def _():
        b = wid
        pltpu.sync_copy(fidx_ref.at[pl.ds(b * F, F)], idx_v)
        pltpu.async_copy(bank_ref.at[idx_v], rows_v, sem).wait()
        pltpu.sync_copy(rows_v, out_ref.at[pl.ds(b * F, F)])


def kernel(basis_vector_bank, task_f, img_f, set_type_indices, w_t1, w_t2,
           w_i1, bn1_g, bn1_b, w_i2, bn2_g, bn2_b, mlp_w1, mlp_b1, ln_g, ln_b,
           mlp_w2, mlp_b2):
    B, F, V, D = basis_vector_bank.shape
    T = NUM_TYPES
    K = min(F, V)

    x = task_f.reshape(task_f.shape[0], task_f.shape[1], -1)
    rms = jnp.sqrt(jnp.mean(x ** 2, axis=(1, 2), keepdims=True))
    x = x / (rms + 1e-8)
    b_, c_, e_ = x.shape
    h = _conv1d(x.reshape(b_, 1, c_ * e_), w_t1)
    h = _leaky_relu(h)
    h = _conv1d(h, w_t2)
    task_emb = jnp.mean(h, axis=2)
    y = _conv2d(img_f, w_i1)
    y = _batchnorm(y, bn1_g, bn1_b)
    y = _leaky_relu(y)
    y = _conv2d(y, w_i2)
    y = _batchnorm(y, bn2_g, bn2_b)
    y = _leaky_relu(y)
    img_emb = jnp.mean(y, axis=(2, 3))
    f = jnp.concatenate([task_emb, img_emb], axis=1)
    f = f @ mlp_w1.T + mlp_b1
    f = _layernorm(f, ln_g, ln_b)
    f = jnp.maximum(f, 0.0)
    filter_f = f @ mlp_w2.T + mlp_b2
    fnorm = jnp.linalg.norm(filter_f, axis=-1, keepdims=True)
    fn = filter_f / jnp.maximum(fnorm, 1e-12)

    sti = set_type_indices.astype(jnp.int32)
    eq = sti[:, :, None] == sti[:, None, :]
    lower = jnp.tril(jnp.ones((F, F), dtype=jnp.int32), -1)
    rank = jnp.sum(eq.astype(jnp.int32) * lower[None, :, :], axis=2
                   ).astype(jnp.int32)
    rep = jnp.argmax(sti[:, None, :] == jnp.arange(T, dtype=jnp.int32)[None, :, None],
                     axis=2).astype(jnp.int32)

    GB = 2
    bank_specs = [
        pl.BlockSpec((1, 1, V, D),
                     functools.partial(
                         lambda i, rep_ref, g=0, t=0:
                             (i * GB + g, rep_ref[i * GB + g, t], 0, 0),
                         g=g, t=t))
        for g in range(GB) for t in range(T)
    ]
    grid_spec = pltpu.PrefetchScalarGridSpec(
        num_scalar_prefetch=1,
        grid=(B // GB,),
        in_specs=[
            pl.BlockSpec((GB, 1, D), lambda i, rep_ref: (i, 0, 0)),
            pl.BlockSpec((GB, 1, F), lambda i, rep_ref: (i, 0, 0)),
            pl.BlockSpec((GB, 1, F), lambda i, rep_ref: (i, 0, 0)),
        ] + bank_specs,
        out_specs=pl.BlockSpec((GB, 1, F), lambda i, rep_ref: (i, 0, 0)),
    )
    fidx = pl.pallas_call(
        functools.partial(_score_topk_kernel, F=F, V=V, D=D, K=K, GB=GB),
        grid_spec=grid_spec,
        out_shape=jax.ShapeDtypeStruct((B, 1, F), jnp.int32),
    )(rep, fn.reshape(B, 1, D), sti.reshape(B, 1, F), rank.reshape(B, 1, F),
      *([basis_vector_bank] * (GB * T)))

    info = plsc.get_sparse_core_info()
    NC = info.num_cores
    mesh = plsc.VectorSubcoreMesh(core_axis_name="c", subcore_axis_name="s")
    sc = pl.kernel(
        functools.partial(_sc_gather_kernel, B=B, F=F, NC=NC),
        mesh=mesh,
        out_type=jax.ShapeDtypeStruct((B * F, D), jnp.float32),
        scratch_types=[
            pltpu.VMEM((F,), jnp.int32),
            pltpu.VMEM((F, D), jnp.float32),
            pltpu.SemaphoreType.DMA,
        ],
    )
    out = sc(basis_vector_bank.reshape(B * F * V, D), fidx.reshape(B * F))
    return out.reshape(B, F, D)


# speedup vs baseline: 1.0315x; 1.0315x over previous
import functools

import jax
import jax.numpy as jnp
from jax import lax
from jax.experimental import pallas as pl
from jax.experimental.pallas import tpu as pltpu
from jax.experimental.pallas import tpu_sc as plsc


def _leaky_relu(x, a=0.2):
    return jnp.where(x >= 0, x, a * x)


def _conv1d(x, w):
    return lax.conv_general_dilated(
        x, w, window_strides=(1,), padding=((1, 1),),
        dimension_numbers=('NCH', 'OIH', 'NCH'))


def _conv2d(x, w):
    return lax.conv_general_dilated(
        x, w, window_strides=(1, 1), padding=((1, 1), (1, 1)),
        dimension_numbers=('NCHW', 'OIHW', 'NCHW'))


def _batchnorm(x, g, b, eps=1e-5):
    m = jnp.mean(x, axis=(0, 2, 3), keepdims=True)
    v = jnp.var(x, axis=(0, 2, 3), keepdims=True)
    return (x - m) / jnp.sqrt(v + eps) * g.reshape(1, -1, 1, 1) + b.reshape(1, -1, 1, 1)


def _layernorm(x, g, b, eps=1e-5):
    m = jnp.mean(x, axis=-1, keepdims=True)
    v = jnp.var(x, axis=-1, keepdims=True)
    return (x - m) / jnp.sqrt(v + eps) * g + b


NUM_TYPES = 4


def _score_topk_kernel(rep_ref, filt_ref, sti_ref, rank_ref, *refs,
                       F, V, D, K, GB):
    T = NUM_TYPES
    G = GB * T
    bank_refs = refs[:G]
    fidx_ref = refs[G]
    i = pl.program_id(0)
    scores = []
    for g in range(GB):
        fv = filt_ref[g, 0]
        fb = fv.astype(jnp.bfloat16).astype(jnp.float32)
        for t in range(T):
            x = bank_refs[g * T + t][0, 0]
            n2 = jnp.sum(x * x, axis=1)
            n = jnp.maximum(jnp.sqrt(n2), 1e-12)
            xb = (x / n[:, None]).astype(jnp.bfloat16).astype(jnp.float32)
            scores.append(jnp.sum(xb * fb[None, :], axis=1))
    R = V // 128
    s4 = jnp.stack(scores).reshape(G, R, 128)
    flat_i = jnp.broadcast_to(
        (lax.broadcasted_iota(jnp.int32, (R, 128), 0) * 128
         + lax.broadcasted_iota(jnp.int32, (R, 128), 1))[None],
        (G, R, 128))
    sti_rows = sti_ref[:, 0]
    rank_rows = rank_ref[:, 0]
    neg_inf = jnp.float32(-jnp.inf)
    big = jnp.int32(2 ** 30)
    maxc = jnp.int32(0)
    for t in range(T):
        cnt = jnp.sum((sti_rows == t).astype(jnp.int32), axis=1)
        maxc = jnp.maximum(maxc, jnp.max(cnt))

    def body(j, carry):
        s4, accs = carry
        m4 = jnp.max(s4, axis=(1, 2), keepdims=True)
        idx4 = jnp.min(jnp.where(s4 == m4, flat_i, big),
                       axis=(1, 2), keepdims=True)
        new_accs = []
        for g in range(GB):
            acc = accs[g]
            sel_j = rank_rows[g:g + 1] == j
            for t in range(T):
                acc = jnp.where(sel_j & (sti_rows[g:g + 1] == t),
                                idx4[g * T + t, 0, 0], acc)
            new_accs.append(acc)
        s4 = jnp.where(flat_i == idx4, neg_inf, s4)
        return s4, tuple(new_accs)

    _, accs = lax.fori_loop(
        0, maxc, body,
        (s4, tuple(jnp.zeros((1, F), jnp.int32) for _ in range(GB))))
    for g in range(GB):
        b = i * GB + g
        base = jnp.zeros((1, F), jnp.int32)
        for t in range(T):
            base = jnp.where(sti_rows[g:g + 1] == t,
                             (b * F + rep_ref[b, t]) * V, base)
        fidx_ref[g] = base + accs[g]


def _sc_gather_kernel(bank_ref, fidx_ref, out_ref, idx_v, rows_v, sem,
                      *, B, F, NC):
    c = lax.axis_index("c")
    s = lax.axis_index("s")
    wid = s * NC + c

    @pl.when(wid < B)
    def _():
        b = wid
        pltpu.sync_copy(fidx_ref.at[pl.ds(b * F, F)], idx_v)
        pltpu.async_copy(bank_ref.at[idx_v], rows_v, sem).wait()
        pltpu.sync_copy(rows_v, out_ref.at[pl.ds(b * F, F)])


def kernel(basis_vector_bank, task_f, img_f, set_type_indices, w_t1, w_t2,
           w_i1, bn1_g, bn1_b, w_i2, bn2_g, bn2_b, mlp_w1, mlp_b1, ln_g, ln_b,
           mlp_w2, mlp_b2):
    B, F, V, D = basis_vector_bank.shape
    T = NUM_TYPES
    K = min(F, V)

    x = task_f.reshape(task_f.shape[0], task_f.shape[1], -1)
    rms = jnp.sqrt(jnp.mean(x ** 2, axis=(1, 2), keepdims=True))
    x = x / (rms + 1e-8)
    b_, c_, e_ = x.shape
    h = _conv1d(x.reshape(b_, 1, c_ * e_), w_t1)
    h = _leaky_relu(h)
    h = _conv1d(h, w_t2)
    task_emb = jnp.mean(h, axis=2)
    y = _conv2d(img_f, w_i1)
    y = _batchnorm(y, bn1_g, bn1_b)
    y = _leaky_relu(y)
    y = _conv2d(y, w_i2)
    y = _batchnorm(y, bn2_g, bn2_b)
    y = _leaky_relu(y)
    img_emb = jnp.mean(y, axis=(2, 3))
    f = jnp.concatenate([task_emb, img_emb], axis=1)
    f = f @ mlp_w1.T + mlp_b1
    f = _layernorm(f, ln_g, ln_b)
    f = jnp.maximum(f, 0.0)
    filter_f = f @ mlp_w2.T + mlp_b2
    fnorm = jnp.linalg.norm(filter_f, axis=-1, keepdims=True)
    fn = filter_f / jnp.maximum(fnorm, 1e-12)

    sti = set_type_indices.astype(jnp.int32)
    eq = sti[:, :, None] == sti[:, None, :]
    lower = jnp.tril(jnp.ones((F, F), dtype=jnp.int32), -1)
    rank = jnp.sum(eq.astype(jnp.int32) * lower[None, :, :], axis=2
                   ).astype(jnp.int32)
    rep = jnp.argmax(sti[:, None, :] == jnp.arange(T, dtype=jnp.int32)[None, :, None],
                     axis=2).astype(jnp.int32)

    GB = 4
    bank_specs = [
        pl.BlockSpec((1, 1, V, D),
                     functools.partial(
                         lambda i, rep_ref, g=0, t=0:
                             (i * GB + g, rep_ref[i * GB + g, t], 0, 0),
                         g=g, t=t))
        for g in range(GB) for t in range(T)
    ]
    grid_spec = pltpu.PrefetchScalarGridSpec(
        num_scalar_prefetch=1,
        grid=(B // GB,),
        in_specs=[
            pl.BlockSpec((GB, 1, D), lambda i, rep_ref: (i, 0, 0)),
            pl.BlockSpec((GB, 1, F), lambda i, rep_ref: (i, 0, 0)),
            pl.BlockSpec((GB, 1, F), lambda i, rep_ref: (i, 0, 0)),
        ] + bank_specs,
        out_specs=pl.BlockSpec((GB, 1, F), lambda i, rep_ref: (i, 0, 0)),
    )
    fidx = pl.pallas_call(
        functools.partial(_score_topk_kernel, F=F, V=V, D=D, K=K, GB=GB),
        grid_spec=grid_spec,
        out_shape=jax.ShapeDtypeStruct((B, 1, F), jnp.int32),
    )(rep, fn.reshape(B, 1, D), sti.reshape(B, 1, F), rank.reshape(B, 1, F),
      *([basis_vector_bank] * (GB * T)))

    info = plsc.get_sparse_core_info()
    NC = info.num_cores
    mesh = plsc.VectorSubcoreMesh(core_axis_name="c", subcore_axis_name="s")
    sc = pl.kernel(
        functools.partial(_sc_gather_kernel, B=B, F=F, NC=NC),
        mesh=mesh,
        out_type=jax.ShapeDtypeStruct((B * F, D), jnp.float32),
        scratch_types=[
            pltpu.VMEM((F,), jnp.int32),
            pltpu.VMEM((F, D), jnp.float32),
            pltpu.SemaphoreType.DMA,
        ],
    )
    out = sc(basis_vector_bank.reshape(B * F * V, D), fidx.reshape(B * F))
    return out.reshape(B, F, D)
